# baseline (speedup 1.0000x reference)
import functools

import jax
import jax.numpy as jnp
from jax import lax
from jax.experimental import pallas as pl
from jax.experimental.pallas import tpu as pltpu
from jax.experimental.pallas import tpu_sc as plsc

B = 16384
CAT = 26
DN = 13
VROWS = 100001
DT = 32
NF = DN + CAT

NC = 2
NS = 16
NW = NC * NS
BPW = B // NW
SB = 16
NSUB = BPW // SB
GROWS = SB * CAT
SROWS = SB * NF
L = 16


def _body(xcat_ref, xnum_ref, w_ref, b_ref, table_ref, out_ref,
          idx_v, pat_v, rows_v, stage_v, xnum_v, w_v, bias_v, sem):
    wid = lax.axis_index("s") * NC + lax.axis_index("c")
    b0 = wid * BPW

    for j in range(DN):
        pltpu.sync_copy(xnum_ref.at[pl.ds(j * B + b0, BPW)],
                        xnum_v.at[pl.ds(j * BPW, BPW)])
    pltpu.sync_copy(w_ref, w_v)
    pltpu.sync_copy(b_ref, bias_v)

    @pl.loop(0, GROWS // L)
    def _build_pat(i):
        k = i * L + lax.iota(jnp.int32, L)
        c = lax.rem(k, jnp.int32(CAT))
        pat_v[pl.ds(i * L, L)] = c * jnp.int32(VROWS)

    @pl.loop(0, NSUB)
    def _subchunk(s):
        bb0 = b0 + s * SB

        pltpu.sync_copy(xcat_ref.at[pl.ds(bb0 * CAT, GROWS)], idx_v)

        @pl.loop(0, GROWS // L)
        def _addoff(i):
            idx_v[pl.ds(i * L, L)] = idx_v[pl.ds(i * L, L)] + pat_v[pl.ds(i * L, L)]

        cps = []
        for g0 in range(0, GROWS, 128):
            gn = min(128, GROWS - g0)
            cps.append(pltpu.async_copy(
                table_ref.at[idx_v.at[pl.ds(g0, gn)]],
                rows_v.at[pl.ds(g0, gn)], sem))
        for cp in cps:
            cp.wait()

        @pl.loop(0, DN)
        def _numeric(j):
            w0 = w_v[pl.ds(j * DT, L)]
            w1 = w_v[pl.ds(j * DT + L, L)]
            a0 = bias_v[pl.ds(j * DT, L)]
            a1 = bias_v[pl.ds(j * DT + L, L)]
            xv = xnum_v[pl.ds(j * BPW + s * SB, L)]
            for b in range(SB):
                v = xv[b]
                r = b * NF + j
                stage_v[r, pl.ds(0, L)] = w0 * v + a0
                stage_v[r, pl.ds(L, L)] = w1 * v + a1

        @pl.loop(0, CAT)
        def _cat(c):
            a0 = bias_v[pl.ds((DN + c) * DT, L)]
            a1 = bias_v[pl.ds((DN + c) * DT + L, L)]
            for b in range(SB):
                g = b * CAT + c
                r = b * NF + DN + c
                stage_v[r, pl.ds(0, L)] = rows_v[g, pl.ds(0, L)] + a0
                stage_v[r, pl.ds(L, L)] = rows_v[g, pl.ds(L, L)] + a1

        pltpu.sync_copy(stage_v, out_ref.at[pl.ds(bb0 * NF, SROWS)])


@jax.jit
def _tokenize(xcat_flat, xnum_flat, w_flat, bias_flat, table2d):
    mesh = plsc.VectorSubcoreMesh(
        core_axis_name="c", subcore_axis_name="s",
        num_cores=NC, num_subcores=NS)
    return pl.kernel(
        _body,
        out_type=jax.ShapeDtypeStruct((B * NF, DT), jnp.float32),
        mesh=mesh,
        scratch_types=[
            pltpu.VMEM((GROWS,), jnp.int32),
            pltpu.VMEM((GROWS,), jnp.int32),
            pltpu.VMEM((GROWS, DT), jnp.float32),
            pltpu.VMEM((SROWS, DT), jnp.float32),
            pltpu.VMEM((DN * BPW,), jnp.float32),
            pltpu.VMEM((DN * DT,), jnp.float32),
            pltpu.VMEM((NF * DT,), jnp.float32),
            pltpu.SemaphoreType.DMA,
        ],
        compiler_params=pltpu.CompilerParams(use_tc_tiling_on_sc=False),
    )(xcat_flat, xnum_flat, w_flat, bias_flat, table2d)


def kernel(x_cat, x_num, weight, bias, tables):
    xcat_flat = x_cat.astype(jnp.int32).reshape(B * CAT)
    xnum_flat = x_num.T.reshape(DN * B)
    w_flat = weight.reshape(DN * DT)
    bias_flat = bias.reshape(NF * DT)
    table2d = tables.reshape(CAT * VROWS, DT)
    out = _tokenize(xcat_flat, xnum_flat, w_flat, bias_flat, table2d)
    return out.reshape(B, NF, DT)


# speedup vs baseline: 20.2985x; 20.2985x over previous
import functools

import jax
import jax.numpy as jnp
from jax import lax
from jax.experimental import pallas as pl
from jax.experimental.pallas import tpu as pltpu
from jax.experimental.pallas import tpu_sc as plsc

B = 16384
CAT = 26
DN = 13
VROWS = 100001
VB = (VROWS // 128) * 128
VT = VROWS - VB
DT = 32
NF = DN + CAT

NC = 2
NS = 16
NW = NC * NS
L = 16
CB = 8192
NH = B // CB


def _body(tab_ref, tail_ref, xcat_ref, xnum_ref, wp_ref, bp_ref, out_ref,
          row_v, tail_v, idx_v, o_v, wb_v, bias_v, sem):
    w = lax.axis_index("s") * NC + lax.axis_index("c")
    lane = lax.iota(jnp.int32, L)

    pltpu.sync_copy(bp_ref.at[pl.ds(w * 128, 128)], bias_v)
    pltpu.sync_copy(wp_ref, wb_v)

    whalf = (w // L) * L
    wmod = w % L

    def splat(vec, f_mod):
        return jnp.sum(jnp.where(lane == f_mod, vec, 0.0))

    for j in range(DN):
        ws = splat(wb_v[pl.ds(j * 128 + whalf, L)], wmod)
        bs = splat(bias_v[pl.ds((j // L) * L, L)], j % L)
        for h in range(NH):
            pltpu.sync_copy(xnum_ref.at[j, pl.ds(h * CB, CB)], o_v)

            @pl.loop(0, CB // L, unroll=8)
            def _num(i):
                xv = o_v[pl.ds(i * L, L)]
                o_v[pl.ds(i * L, L)] = ws * xv + bs

            pltpu.sync_copy(o_v, out_ref.at[j * DT + w, pl.ds(h * CB, CB)])

    @pl.loop(0, CAT)
    def _cat(c):
        pltpu.sync_copy(tab_ref.at[c * DT + w, pl.ds(0, VB)], row_v)
        pltpu.sync_copy(tail_ref.at[pl.ds((c * DT + w) * 128, 128)], tail_v)
        f = DN + c
        bs = splat(bias_v[pl.ds((f // L) * L, L)], f % L)
        for h in range(NH):
            pltpu.sync_copy(xcat_ref.at[c, pl.ds(h * CB, CB)], idx_v)

            @pl.loop(0, CB // L, unroll=8)
            def _gather(i):
                ii = idx_v[pl.ds(i * L, L)]
                in_tail = ii >= VB
                g0 = plsc.load_gather(row_v, [jnp.minimum(ii, VB - 1)])
                g1 = plsc.load_gather(tail_v, [jnp.maximum(ii - VB, 0)])
                o_v[pl.ds(i * L, L)] = jnp.where(in_tail, g1, g0) + bs

            pltpu.sync_copy(o_v, out_ref.at[f * DT + w, pl.ds(h * CB, CB)])


@jax.jit
def _tokenize(tabT, tab_tail, xcatT, xnumT, weightP, biasP):
    mesh = plsc.VectorSubcoreMesh(
        core_axis_name="c", subcore_axis_name="s",
        num_cores=NC, num_subcores=NS)
    return pl.kernel(
        _body,
        out_type=jax.ShapeDtypeStruct((NF * DT, B), jnp.float32),
        mesh=mesh,
        scratch_types=[
            pltpu.VMEM((VB,), jnp.float32),
            pltpu.VMEM((128,), jnp.float32),
            pltpu.VMEM((CB,), jnp.int32),
            pltpu.VMEM((CB,), jnp.float32),
            pltpu.VMEM((DN * 128,), jnp.float32),
            pltpu.VMEM((128,), jnp.float32),
            pltpu.SemaphoreType.DMA,
        ],
        compiler_params=pltpu.CompilerParams(
            use_tc_tiling_on_sc=True, needs_layout_passes=False),
    )(tabT, tab_tail, xcatT, xnumT, weightP, biasP)


def kernel(x_cat, x_num, weight, bias, tables):
    tabT = tables.transpose(0, 2, 1).reshape(CAT * DT, VROWS)
    xcatT = x_cat.astype(jnp.int32).T
    xnumT = jnp.pad(x_num.T, ((0, 3), (0, 0)))
    tab_tail = jnp.pad(tabT[:, VB:], ((0, 0), (0, 128 - VT))).reshape(-1)
    weightP = jnp.pad(weight, ((0, 0), (0, 128 - DT))).reshape(-1)
    biasP = jnp.pad(bias.T, ((0, 0), (0, 128 - NF))).reshape(-1)
    out = _tokenize(tabT, tab_tail, xcatT, xnumT, weightP, biasP)
    return out.reshape(NF, DT, B).transpose(2, 0, 1)


# speedup vs baseline: 24.6808x; 1.2159x over previous
import functools

import jax
import jax.numpy as jnp
from jax import lax
from jax.experimental import pallas as pl
from jax.experimental.pallas import tpu as pltpu
from jax.experimental.pallas import tpu_sc as plsc

B = 16384
CAT = 26
DN = 13
VROWS = 100001
VB = (VROWS // 128) * 128
VT = VROWS - VB
DT = 32
NF = DN + CAT

NC = 2
NS = 16
NW = NC * NS
L = 16
CB = 8192
NH = B // CB


def _body(tab_ref, tail_ref, xcat_ref, xnum_ref, wp_ref, bp_ref, out_ref,
          row_v, idx_v, o_v, wb_v, bias_v, sem_r, sem_t, sem_i):
    w = lax.axis_index("s") * NC + lax.axis_index("c")
    lane = lax.iota(jnp.int32, L)

    pltpu.sync_copy(bp_ref.at[pl.ds(w * 128, 128)], bias_v)
    pltpu.sync_copy(wp_ref, wb_v)

    whalf = (w // L) * L
    wmod = w % L

    def splat(vec, f_mod):
        return jnp.sum(jnp.where(lane == f_mod, vec, 0.0))

    for j in range(DN):
        ws = splat(wb_v[pl.ds(j * 128 + whalf, L)], wmod)
        bs = splat(bias_v[pl.ds((j // L) * L, L)], j % L)
        for h in range(NH):
            pltpu.sync_copy(xnum_ref.at[j, pl.ds(h * CB, CB)], o_v)

            @pl.loop(0, CB // L, unroll=8)
            def _num(i):
                xv = o_v[pl.ds(i * L, L)]
                o_v[pl.ds(i * L, L)] = ws * xv + bs

            pltpu.sync_copy(o_v, out_ref.at[j * DT + w, pl.ds(h * CB, CB)])

    @pl.loop(0, CAT)
    def _cat(c):
        r = c * DT + w
        rd = pltpu.async_copy(tab_ref.at[r, pl.ds(0, VB)],
                              row_v.at[pl.ds(0, VB)], sem_r)
        td = pltpu.async_copy(tail_ref.at[pl.ds(r * 128, 128)],
                              row_v.at[pl.ds(VB, 128)], sem_t)
        xd = pltpu.async_copy(xcat_ref.at[c, pl.ds(0, B)], idx_v, sem_i)
        f = DN + c
        bs = splat(bias_v[pl.ds((f // L) * L, L)], f % L)
        rd.wait()
        td.wait()
        xd.wait()
        for h in range(NH):

            @pl.loop(0, CB // L, unroll=8)
            def _gather(i):
                ii = idx_v[pl.ds(h * CB + i * L, L)]
                g = plsc.load_gather(row_v, [ii])
                o_v[pl.ds(i * L, L)] = g + bs

            pltpu.sync_copy(o_v, out_ref.at[f * DT + w, pl.ds(h * CB, CB)])


@jax.jit
def _tokenize(tabT, tab_tail, xcatT, xnumT, weightP, biasP):
    mesh = plsc.VectorSubcoreMesh(
        core_axis_name="c", subcore_axis_name="s",
        num_cores=NC, num_subcores=NS)
    return pl.kernel(
        _body,
        out_type=jax.ShapeDtypeStruct((NF * DT, B), jnp.float32),
        mesh=mesh,
        scratch_types=[
            pltpu.VMEM((VB + 128,), jnp.float32),
            pltpu.VMEM((B,), jnp.int32),
            pltpu.VMEM((CB,), jnp.float32),
            pltpu.VMEM((DN * 128,), jnp.float32),
            pltpu.VMEM((128,), jnp.float32),
            pltpu.SemaphoreType.DMA,
            pltpu.SemaphoreType.DMA,
            pltpu.SemaphoreType.DMA,
        ],
        compiler_params=pltpu.CompilerParams(
            use_tc_tiling_on_sc=True, needs_layout_passes=False),
    )(tabT, tab_tail, xcatT, xnumT, weightP, biasP)


def kernel(x_cat, x_num, weight, bias, tables):
    tabT = tables.transpose(0, 2, 1).reshape(CAT * DT, VROWS)
    xcatT = x_cat.astype(jnp.int32).T
    xnumT = jnp.pad(x_num.T, ((0, 3), (0, 0)))
    tab_tail = jnp.pad(tabT[:, VB:], ((0, 0), (0, 128 - VT))).reshape(-1)
    weightP = jnp.pad(weight, ((0, 0), (0, 128 - DT))).reshape(-1)
    biasP = jnp.pad(bias.T, ((0, 0), (0, 128 - NF))).reshape(-1)
    out = _tokenize(tabT, tab_tail, xcatT, xnumT, weightP, biasP)
    return out.reshape(NF, DT, B).transpose(2, 0, 1)


# speedup vs baseline: 25.9815x; 1.0527x over previous
import functools

import jax
import jax.numpy as jnp
from jax import lax
from jax.experimental import pallas as pl
from jax.experimental.pallas import tpu as pltpu
from jax.experimental.pallas import tpu_sc as plsc

B = 16384
CAT = 26
DN = 13
VROWS = 100001
VB = (VROWS // 128) * 128
VT = VROWS - VB
DT = 32
NF = DN + CAT

NC = 2
NS = 16
NW = NC * NS
L = 16
CQ = 4096
NQ = B // CQ
_T4 = (VB // 128) // 4
ROW_CUTS = [0, (_T4 + 1) * 128, (2 * _T4 + 1) * 128, (3 * _T4 + 1) * 128, VB]


def _body(tab_ref, tail_ref, xcat_ref, xnum_ref, wp_ref, bp_ref, out_ref,
          row_v, idx_v, ob0, ob1, wb_v, bias_v, sem_r, sem_t, sem_i, sem_o):
    w = lax.axis_index("s") * NC + lax.axis_index("c")
    lane = lax.iota(jnp.int32, L)
    obufs = (ob0, ob1)

    pltpu.sync_copy(bp_ref.at[pl.ds(w * 128, 128)], bias_v)
    pltpu.sync_copy(wp_ref, wb_v)

    whalf = (w // L) * L
    wmod = w % L

    def splat(vec, f_mod):
        return jnp.sum(jnp.where(lane == f_mod, vec, 0.0))

    n_out = 0

    for g0 in (0, 6, 12):
        n = min(6, DN - g0)
        loads = [
            pltpu.async_copy(xnum_ref.at[g0 + k, pl.ds(0, B)],
                             row_v.at[pl.ds(k * B, B)], sem_i)
            for k in range(n)
        ]
        for ld in loads:
            ld.wait()
        for k in range(n):
            j = g0 + k
            ws = splat(wb_v[pl.ds(j * 128 + whalf, L)], wmod)
            bs = splat(bias_v[pl.ds((j // L) * L, L)], j % L)
            for q in range(NQ):
                p = obufs[(j * NQ + q) % 2]
                if n_out >= 2:
                    pltpu.make_async_copy(
                        p, out_ref.at[0, pl.ds(0, CQ)], sem_o).wait()
                    n_out -= 1

                @pl.loop(0, CQ // L, unroll=8)
                def _num(i):
                    xv = row_v[pl.ds(k * B + q * CQ + i * L, L)]
                    p[pl.ds(i * L, L)] = ws * xv + bs

                pltpu.async_copy(
                    p, out_ref.at[j * DT + w, pl.ds(q * CQ, CQ)], sem_o)
                n_out += 1

    pltpu.async_copy(xcat_ref.at[0, pl.ds(0, B)], idx_v, sem_i)

    @pl.loop(0, CAT)
    def _cat(c):
        r = c * DT + w
        rds = [
            pltpu.async_copy(
                tab_ref.at[r, pl.ds(ROW_CUTS[k], ROW_CUTS[k + 1] - ROW_CUTS[k])],
                row_v.at[pl.ds(ROW_CUTS[k], ROW_CUTS[k + 1] - ROW_CUTS[k])],
                sem_r)
            for k in range(4)
        ]
        td = pltpu.async_copy(tail_ref.at[pl.ds(r * 128, 128)],
                              row_v.at[pl.ds(VB, 128)], sem_t)
        f = DN + c
        bs = splat(bias_v[pl.ds((f // L) * L, L)], f % L)
        pltpu.make_async_copy(xcat_ref.at[c, pl.ds(0, B)], idx_v, sem_i).wait()
        for rd in rds:
            rd.wait()
        td.wait()
        for q in range(NQ):
            p = obufs[q % 2]
            pltpu.make_async_copy(p, out_ref.at[0, pl.ds(0, CQ)], sem_o).wait()

            @pl.loop(0, CQ // L, unroll=8)
            def _gather(i):
                ii = idx_v[pl.ds(q * CQ + i * L, L)]
                g = plsc.load_gather(row_v, [ii])
                p[pl.ds(i * L, L)] = g + bs

            pltpu.async_copy(p, out_ref.at[f * DT + w, pl.ds(q * CQ, CQ)],
                             sem_o)

        @pl.when(c + 1 < CAT)
        def _prefetch():
            pltpu.async_copy(xcat_ref.at[c + 1, pl.ds(0, B)], idx_v, sem_i)

    for _ in range(2):
        pltpu.make_async_copy(ob0, out_ref.at[0, pl.ds(0, CQ)], sem_o).wait()


@jax.jit
def _tokenize(tabT, tab_tail, xcatT, xnumT, weightP, biasP):
    mesh = plsc.VectorSubcoreMesh(
        core_axis_name="c", subcore_axis_name="s",
        num_cores=NC, num_subcores=NS)
    return pl.kernel(
        _body,
        out_type=jax.ShapeDtypeStruct((NF * DT, B), jnp.float32),
        mesh=mesh,
        scratch_types=[
            pltpu.VMEM((VB + 128,), jnp.float32),
            pltpu.VMEM((B,), jnp.int32),
            pltpu.VMEM((CQ,), jnp.float32),
            pltpu.VMEM((CQ,), jnp.float32),
            pltpu.VMEM((DN * 128,), jnp.float32),
            pltpu.VMEM((128,), jnp.float32),
            pltpu.SemaphoreType.DMA,
            pltpu.SemaphoreType.DMA,
            pltpu.SemaphoreType.DMA,
            pltpu.SemaphoreType.DMA,
        ],
        compiler_params=pltpu.CompilerParams(
            use_tc_tiling_on_sc=True, needs_layout_passes=False),
    )(tabT, tab_tail, xcatT, xnumT, weightP, biasP)


def kernel(x_cat, x_num, weight, bias, tables):
    tabT = tables.transpose(0, 2, 1).reshape(CAT * DT, VROWS)
    xcatT = x_cat.astype(jnp.int32).T
    xnumT = jnp.pad(x_num.T, ((0, 3), (0, 0)))
    tab_tail = jnp.pad(tabT[:, VB:], ((0, 0), (0, 128 - VT))).reshape(-1)
    weightP = jnp.pad(weight, ((0, 0), (0, 128 - DT))).reshape(-1)
    biasP = jnp.pad(bias.T, ((0, 0), (0, 128 - NF))).reshape(-1)
    out = _tokenize(tabT, tab_tail, xcatT, xnumT, weightP, biasP)
    return out.reshape(NF, DT, B).transpose(2, 0, 1)


# speedup vs baseline: 25.9900x; 1.0003x over previous
import functools

import jax
import jax.numpy as jnp
from jax import lax
from jax.experimental import pallas as pl
from jax.experimental.pallas import tpu as pltpu
from jax.experimental.pallas import tpu_sc as plsc

B = 16384
CAT = 26
DN = 13
VROWS = 100001
VB = (VROWS // 128) * 128
VT = VROWS - VB
DT = 32
NF = DN + CAT

NC = 2
NS = 16
NW = NC * NS
L = 16
CQ = 4096
NQ = B // CQ
_T4 = (VB // 128) // 4
ROW_CUTS = [0, (_T4 + 1) * 128, (2 * _T4 + 1) * 128, (3 * _T4 + 1) * 128, VB]


def _body(tab_ref, tail_ref, xcat_ref, xnum_ref, wp_ref, bp_ref, out_ref,
          row_v, idx_v, ob0, ob1, wb_v, bias_v, sem_r, sem_t, sem_i, sem_o):
    w = lax.axis_index("s") * NC + lax.axis_index("c")
    lane = lax.iota(jnp.int32, L)
    obufs = (ob0, ob1)

    pltpu.sync_copy(bp_ref.at[pl.ds(w * 128, 128)], bias_v)
    pltpu.sync_copy(wp_ref, wb_v)

    whalf = (w // L) * L
    wmod = w % L

    def splat(vec, f_mod):
        return jnp.sum(jnp.where(lane == f_mod, vec, 0.0))

    n_out = 0

    for g0 in (0, 6, 12):
        n = min(6, DN - g0)
        loads = [
            pltpu.async_copy(xnum_ref.at[g0 + k, pl.ds(0, B)],
                             row_v.at[pl.ds(k * B, B)], sem_i)
            for k in range(n)
        ]
        for ld in loads:
            ld.wait()
        for k in range(n):
            j = g0 + k
            ws = splat(wb_v[pl.ds(j * 128 + whalf, L)], wmod)
            bs = splat(bias_v[pl.ds((j // L) * L, L)], j % L)
            for q in range(NQ):
                p = obufs[(j * NQ + q) % 2]
                if n_out >= 2:
                    pltpu.make_async_copy(
                        p, out_ref.at[0, pl.ds(0, CQ)], sem_o).wait()
                    n_out -= 1

                @pl.loop(0, CQ // L, unroll=16)
                def _num(i):
                    xv = row_v[pl.ds(k * B + q * CQ + i * L, L)]
                    p[pl.ds(i * L, L)] = ws * xv + bs

                pltpu.async_copy(
                    p, out_ref.at[j * DT + w, pl.ds(q * CQ, CQ)], sem_o)
                n_out += 1

    pltpu.async_copy(xcat_ref.at[0, pl.ds(0, B)], idx_v, sem_i)

    @pl.loop(0, CAT)
    def _cat(c):
        r = c * DT + w
        rds = [
            pltpu.async_copy(
                tab_ref.at[r, pl.ds(ROW_CUTS[k], ROW_CUTS[k + 1] - ROW_CUTS[k])],
                row_v.at[pl.ds(ROW_CUTS[k], ROW_CUTS[k + 1] - ROW_CUTS[k])],
                sem_r)
            for k in range(4)
        ]
        td = pltpu.async_copy(tail_ref.at[pl.ds(r * 128, 128)],
                              row_v.at[pl.ds(VB, 128)], sem_t)
        f = DN + c
        bs = splat(bias_v[pl.ds((f // L) * L, L)], f % L)
        pltpu.make_async_copy(xcat_ref.at[c, pl.ds(0, B)], idx_v, sem_i).wait()
        for rd in rds:
            rd.wait()
        td.wait()
        for q in range(NQ):
            p = obufs[q % 2]
            pltpu.make_async_copy(p, out_ref.at[0, pl.ds(0, CQ)], sem_o).wait()

            @pl.loop(0, CQ // L, unroll=16)
            def _gather(i):
                ii = idx_v[pl.ds(q * CQ + i * L, L)]
                g = plsc.load_gather(row_v, [ii])
                p[pl.ds(i * L, L)] = g + bs

            pltpu.async_copy(p, out_ref.at[f * DT + w, pl.ds(q * CQ, CQ)],
                             sem_o)

        @pl.when(c + 1 < CAT)
        def _prefetch():
            pltpu.async_copy(xcat_ref.at[c + 1, pl.ds(0, B)], idx_v, sem_i)

    for _ in range(2):
        pltpu.make_async_copy(ob0, out_ref.at[0, pl.ds(0, CQ)], sem_o).wait()


@jax.jit
def _tokenize(tabT, tab_tail, xcatT, xnumT, weightP, biasP):
    mesh = plsc.VectorSubcoreMesh(
        core_axis_name="c", subcore_axis_name="s",
        num_cores=NC, num_subcores=NS)
    return pl.kernel(
        _body,
        out_type=jax.ShapeDtypeStruct((NF * DT, B), jnp.float32),
        mesh=mesh,
        scratch_types=[
            pltpu.VMEM((VB + 128,), jnp.float32),
            pltpu.VMEM((B,), jnp.int32),
            pltpu.VMEM((CQ,), jnp.float32),
            pltpu.VMEM((CQ,), jnp.float32),
            pltpu.VMEM((DN * 128,), jnp.float32),
            pltpu.VMEM((128,), jnp.float32),
            pltpu.SemaphoreType.DMA,
            pltpu.SemaphoreType.DMA,
            pltpu.SemaphoreType.DMA,
            pltpu.SemaphoreType.DMA,
        ],
        compiler_params=pltpu.CompilerParams(
            use_tc_tiling_on_sc=True, needs_layout_passes=False),
    )(tabT, tab_tail, xcatT, xnumT, weightP, biasP)


def kernel(x_cat, x_num, weight, bias, tables):
    tabT = tables.transpose(0, 2, 1).reshape(CAT * DT, VROWS)
    xcatT = x_cat.astype(jnp.int32).T
    xnumT = jnp.pad(x_num.T, ((0, 3), (0, 0)))
    tab_tail = jnp.pad(tabT[:, VB:], ((0, 0), (0, 128 - VT))).reshape(-1)
    weightP = jnp.pad(weight, ((0, 0), (0, 128 - DT))).reshape(-1)
    biasP = jnp.pad(bias.T, ((0, 0), (0, 128 - NF))).reshape(-1)
    out = _tokenize(tabT, tab_tail, xcatT, xnumT, weightP, biasP)
    return out.reshape(NF, DT, B).transpose(2, 0, 1)


# speedup vs baseline: 26.0038x; 1.0005x over previous
import jax
import jax.numpy as jnp
from jax import lax
from jax.experimental import pallas as pl
from jax.experimental.pallas import tpu as pltpu
from jax.experimental.pallas import tpu_sc as plsc

B = 16384
CAT = 26
DN = 13
VROWS = 100001
VB = (VROWS // 128) * 128
VT = VROWS - VB
DT = 32
NF = DN + CAT

NC = 2
NS = 16
NW = NC * NS
L = 16
CQ = 4096
NQ = B // CQ
_T4 = (VB // 128) // 4
ROW_CUTS = [0, (_T4 + 1) * 128, (2 * _T4 + 1) * 128, (3 * _T4 + 1) * 128, VB]


def _body(tab_ref, tail_ref, xcat_ref, xnum_ref, wp_ref, bp_ref, out_ref,
          row_v, idx_v, ob0, ob1, wb_v, bias_v, sem_r, sem_t, sem_i, sem_o):
    w = lax.axis_index("s") * NC + lax.axis_index("c")
    lane = lax.iota(jnp.int32, L)
    obufs = (ob0, ob1)

    pltpu.sync_copy(bp_ref.at[pl.ds(w * 128, 128)], bias_v)
    pltpu.sync_copy(wp_ref, wb_v)

    whalf = (w // L) * L
    wmod = w % L

    def splat(vec, f_mod):
        return jnp.sum(jnp.where(lane == f_mod, vec, 0.0))

    n_out = 0

    for g0 in (0, 6, 12):
        n = min(6, DN - g0)
        loads = [
            pltpu.async_copy(xnum_ref.at[g0 + k, pl.ds(0, B)],
                             row_v.at[pl.ds(k * B, B)], sem_i)
            for k in range(n)
        ]
        for ld in loads:
            ld.wait()
        for k in range(n):
            j = g0 + k
            ws = splat(wb_v[pl.ds(j * 128 + whalf, L)], wmod)
            bs = splat(bias_v[pl.ds((j // L) * L, L)], j % L)
            for q in range(NQ):
                p = obufs[(j * NQ + q) % 2]
                if n_out >= 2:
                    pltpu.make_async_copy(
                        p, out_ref.at[0, pl.ds(0, CQ)], sem_o).wait()
                    n_out -= 1

                @pl.loop(0, CQ // L, unroll=16)
                def _num(i):
                    xv = row_v[pl.ds(k * B + q * CQ + i * L, L)]
                    p[pl.ds(i * L, L)] = ws * xv + bs

                pltpu.async_copy(
                    p, out_ref.at[j * DT + w, pl.ds(q * CQ, CQ)], sem_o)
                n_out += 1

    pltpu.async_copy(xcat_ref.at[0, pl.ds(0, B)], idx_v, sem_i)

    @pl.loop(0, CAT)
    def _cat(c):
        r = c * DT + w
        rds = [
            pltpu.async_copy(
                tab_ref.at[r, pl.ds(ROW_CUTS[k], ROW_CUTS[k + 1] - ROW_CUTS[k])],
                row_v.at[pl.ds(ROW_CUTS[k], ROW_CUTS[k + 1] - ROW_CUTS[k])],
                sem_r)
            for k in range(4)
        ]
        td = pltpu.async_copy(tail_ref.at[pl.ds(r * 128, 128)],
                              row_v.at[pl.ds(VB, 128)], sem_t)
        f = DN + c
        bs = splat(bias_v[pl.ds((f // L) * L, L)], f % L)
        pltpu.make_async_copy(xcat_ref.at[c, pl.ds(0, B)], idx_v, sem_i).wait()
        for rd in rds:
            rd.wait()
        td.wait()
        for q in range(NQ):
            p = obufs[q % 2]
            pltpu.make_async_copy(p, out_ref.at[0, pl.ds(0, CQ)], sem_o).wait()

            @pl.loop(0, CQ // L, unroll=16)
            def _gather(i):
                ii = idx_v[pl.ds(q * CQ + i * L, L)]
                g = plsc.load_gather(row_v, [ii])
                p[pl.ds(i * L, L)] = g + bs

            pltpu.async_copy(p, out_ref.at[f * DT + w, pl.ds(q * CQ, CQ)],
                             sem_o)

        @pl.when(c + 1 < CAT)
        def _prefetch():
            pltpu.async_copy(xcat_ref.at[c + 1, pl.ds(0, B)], idx_v, sem_i)

    for _ in range(2):
        pltpu.make_async_copy(ob0, out_ref.at[0, pl.ds(0, CQ)], sem_o).wait()


@jax.jit
def _tokenize(tabT, tab_tail, xcatT, xnumT, weightP, biasP):
    mesh = plsc.VectorSubcoreMesh(
        core_axis_name="c", subcore_axis_name="s",
        num_cores=NC, num_subcores=NS)
    return pl.kernel(
        _body,
        out_type=jax.ShapeDtypeStruct((NF * DT, B), jnp.float32),
        mesh=mesh,
        scratch_types=[
            pltpu.VMEM((VB + 128,), jnp.float32),
            pltpu.VMEM((B,), jnp.int32),
            pltpu.VMEM((CQ,), jnp.float32),
            pltpu.VMEM((CQ,), jnp.float32),
            pltpu.VMEM((DN * 128,), jnp.float32),
            pltpu.VMEM((128,), jnp.float32),
            pltpu.SemaphoreType.DMA,
            pltpu.SemaphoreType.DMA,
            pltpu.SemaphoreType.DMA,
            pltpu.SemaphoreType.DMA,
        ],
        compiler_params=pltpu.CompilerParams(
            use_tc_tiling_on_sc=True, needs_layout_passes=False),
    )(tabT, tab_tail, xcatT, xnumT, weightP, biasP)


def kernel(x_cat, x_num, weight, bias, tables):
    tabT = tables.transpose(0, 2, 1).reshape(CAT * DT, VROWS)
    xcatT = x_cat.astype(jnp.int32).T
    xnumT = jnp.pad(x_num.T, ((0, 3), (0, 0)))
    tab_tail = jnp.pad(tabT[:, VB:], ((0, 0), (0, 128 - VT))).reshape(-1)
    weightP = jnp.pad(weight, ((0, 0), (0, 128 - DT))).reshape(-1)
    biasP = jnp.pad(bias.T, ((0, 0), (0, 128 - NF))).reshape(-1)
    out = _tokenize(tabT, tab_tail, xcatT, xnumT, weightP, biasP)
    return out.reshape(NF, DT, B).transpose(2, 0, 1)
